# baseline (speedup 1.0000x reference)
import functools

import jax
import jax.numpy as jnp
from jax import lax
from jax.experimental import pallas as pl
from jax.experimental.pallas import tpu as pltpu
from jax.experimental.pallas import tpu_sc as plsc

B = 4
N = 2048
C = 67
PD = 3
K = 16
R2 = 4.0
_KEY_R2 = 0x40800000
QB = 256
NB = N // QB
CO = 259


def _knn_body(p0_ref, p1_ref, idx_ref, d2v_ref, *, base):
    p0 = p0_ref[0]
    p1 = p1_ref[0]
    s0 = jnp.sum(p0 * p0, axis=0, keepdims=True)
    s1 = jnp.sum(p1 * p1, axis=1, keepdims=True)
    cross = lax.dot_general(
        p1.astype(jnp.bfloat16), p0.astype(jnp.bfloat16),
        (((1,), (0,)), ((), ())), preferred_element_type=jnp.float32)
    d2 = s1 + s0 - 2.0 * cross
    bits = lax.bitcast_convert_type(d2, jnp.int32)
    iota_j = lax.broadcasted_iota(jnp.int32, (N, QB), 0)
    mkey = bits ^ (jnp.right_shift(bits, 31) & jnp.int32(0x7FFFFFFF))
    key = (mkey & jnp.int32(~2047)) | iota_j
    q = N // 4
    s0, s1, s2, s3 = (key[i * q:(i + 1) * q] for i in range(4))
    s0, s1 = jnp.minimum(s0, s1), jnp.maximum(s0, s1)
    s2, s3 = jnp.minimum(s2, s3), jnp.maximum(s2, s3)
    s0, s2 = jnp.minimum(s0, s2), jnp.maximum(s0, s2)
    s1, s3 = jnp.minimum(s1, s3), jnp.maximum(s1, s3)
    s1, s2 = jnp.minimum(s1, s2), jnp.maximum(s1, s2)
    last = jnp.full((1, QB), jnp.iinfo(jnp.int32).min, jnp.int32)
    big = jnp.iinfo(jnp.int32).max
    for k in range(K):
        cand = jnp.where(s3 > last, s3, big)
        cand = jnp.where(s2 > last, s2, cand)
        cand = jnp.where(s1 > last, s1, cand)
        cand = jnp.where(s0 > last, s0, cand)
        m = jnp.min(cand, axis=0, keepdims=True)
        idx_ref[0, pl.ds(k, 1), :] = (m & jnp.int32(2047)) + base
        d2v_ref[0, pl.ds(k, 1), :] = m
        last = m


def _knn(p0pad, p1pad, b):
    return pl.pallas_call(
        functools.partial(_knn_body, base=b * N),
        grid=(NB,),
        in_specs=[
            pl.BlockSpec((1, 8, QB), lambda i: (0, 0, i)),
            pl.BlockSpec((1, N, 8), lambda i: (0, 0, 0)),
        ],
        out_specs=[
            pl.BlockSpec((1, K, QB), lambda i: (i, 0, 0)),
            pl.BlockSpec((1, K, QB), lambda i: (i, 0, 0)),
        ],
        out_shape=[
            jax.ShapeDtypeStruct((NB, K, QB), jnp.int32),
            jax.ShapeDtypeStruct((NB, K, QB), jnp.int32),
        ],
    )(p0pad, p1pad)


def _a1_body(t_ref, w_ref, o_ref):
    o_ref[...] = jnp.dot(t_ref[...], w_ref[...],
                         preferred_element_type=jnp.float32)


def _a1(table, w1b):
    return pl.pallas_call(
        _a1_body,
        grid=(1,),
        in_specs=[pl.BlockSpec((B * N, 80), lambda i: (0, 0)),
                  pl.BlockSpec((80, 128), lambda i: (0, 0))],
        out_specs=pl.BlockSpec((B * N, 128), lambda i: (0, 0)),
        out_shape=jax.ShapeDtypeStruct((B * N, 128), jnp.float32),
    )(table, w1b)


_NC = 2
_NS = 16
_NW = _NC * _NS
_BTOT = N * K
_BPW = _BTOT // _NW
_GCH = 256


def _sc_gather(table, idxf):
    mesh = plsc.VectorSubcoreMesh(core_axis_name="c", subcore_axis_name="s")

    @functools.partial(
        pl.kernel, mesh=mesh,
        out_type=jax.ShapeDtypeStruct((_BTOT, 128), jnp.float32),
        scratch_types=[
            pltpu.VMEM((_BPW,), jnp.int32),
            pltpu.VMEM((_GCH, 128), jnp.float32),
            pltpu.VMEM((_GCH, 128), jnp.float32),
            pltpu.SemaphoreType.DMA,
            pltpu.SemaphoreType.DMA,
        ])
    def gk(table_hbm, idx_hbm, out_hbm, idx_v, buf0, buf1, sem0, sem1):
        wid = lax.axis_index("s") * _NC + lax.axis_index("c")
        base = wid * _BPW
        pltpu.sync_copy(idx_hbm.at[pl.ds(base, _BPW)], idx_v)
        bufs = (buf0, buf1)
        sems = (sem0, sem1)
        nch = _BPW // _GCH
        for c in range(nch):
            pltpu.make_async_copy(
                table_hbm.at[idx_v.at[pl.ds(c * _GCH, _GCH)]],
                bufs[c % 2], sems[c % 2]).start()
            if c >= 1:
                pltpu.make_async_copy(
                    table_hbm.at[idx_v.at[pl.ds((c - 1) * _GCH, _GCH)]],
                    bufs[(c - 1) % 2], sems[(c - 1) % 2]).wait()
                pltpu.sync_copy(bufs[(c - 1) % 2],
                                out_hbm.at[pl.ds(base + (c - 1) * _GCH, _GCH)])
        pltpu.make_async_copy(
            table_hbm.at[idx_v.at[pl.ds((nch - 1) * _GCH, _GCH)]],
            bufs[(nch - 1) % 2], sems[(nch - 1) % 2]).wait()
        pltpu.sync_copy(bufs[(nch - 1) % 2],
                        out_hbm.at[pl.ds(base + (nch - 1) * _GCH, _GCH)])

    return gk(table, idxf)


def _mlp_body(prev_ref, gth_ref, pts0_ref, p0_ref, d2v_ref, w1a_ref, w2_ref,
              w3_ref, b1_ref, b2_ref, b3_ref, out_ref):
    del prev_ref
    g = gth_ref[...]
    pts0 = pts0_ref[...]
    a0 = jnp.dot(pts0, w1a_ref[...], preferred_element_type=jnp.float32)
    d2v = jnp.transpose(d2v_ref[0], (1, 0))
    b1 = b1_ref[...]
    h1 = jnp.concatenate(
        [jax.nn.relu(g[k * QB:(k + 1) * QB] + a0 + b1) for k in range(K)]
    ).astype(jnp.bfloat16)
    h2 = jax.nn.relu(jnp.dot(h1, w2_ref[...], preferred_element_type=jnp.float32)
                     + b2_ref[...]).astype(jnp.bfloat16)
    h3 = jax.nn.relu(jnp.dot(h2, w3_ref[...], preferred_element_type=jnp.float32)
                     + b3_ref[...])
    acc = jnp.zeros((QB, 256), jnp.float32)
    for k in range(K):
        mask = d2v[:, k:k + 1] >= _KEY_R2
        acc = jnp.maximum(acc, jnp.where(mask, 0.0, h3[k * QB:(k + 1) * QB]))
    out_ref[0, :PD, :] = p0_ref[0, :PD, :]
    out_ref[0, PD:, :] = jnp.transpose(acc, (1, 0))


def _mlp(prev, b, gth, pts0p, p0pad, d2v, w1a, w2t, w3t, b1r, b2r, b3r):
    wspec = lambda shape: pl.BlockSpec(shape, lambda i: (0, 0))
    return pl.pallas_call(
        _mlp_body,
        grid=(NB,),
        in_specs=[
            pl.BlockSpec(memory_space=pl.ANY),
            pl.BlockSpec((K * QB, 128), lambda i: (i, 0)),
            pl.BlockSpec((QB, 72), lambda i: (i, 0)),
            pl.BlockSpec((1, 8, QB), lambda i: (0, 0, i)),
            pl.BlockSpec((1, K, QB), lambda i: (i, 0, 0)),
            wspec((72, 128)), wspec((128, 128)),
            wspec((128, 256)), wspec((1, 128)), wspec((1, 128)),
            wspec((1, 256)),
        ],
        out_specs=pl.BlockSpec((1, CO, QB), lambda i, _b=b: (_b, 0, i)),
        out_shape=jax.ShapeDtypeStruct((B, CO, N), jnp.float32),
        input_output_aliases={0: 0},
    )(prev, gth, pts0p, p0pad, d2v, w1a, w2t, w3t, b1r, b2r, b3r)


def kernel(clouds0, clouds1, W1, b1, W2, b2, W3, b3):
    f32 = jnp.float32
    bf16 = jnp.bfloat16
    p0pad = jnp.pad(clouds0[:, :PD, :], ((0, 0), (0, 8 - PD), (0, 0)))
    p1pad = jnp.pad(jnp.transpose(clouds1[:, :PD, :], (0, 2, 1)),
                    ((0, 0), (0, 0), (0, 8 - PD)))

    pts1 = jnp.transpose(clouds1, (0, 2, 1)).reshape(B * N, C)
    pts1p = jnp.pad(pts1, ((0, 0), (0, 80 - C))).astype(bf16)
    pts0 = jnp.transpose(clouds0, (0, 2, 1)).reshape(B * N, C)
    pts0p = jnp.pad(pts0, ((0, 0), (0, 72 - C))).astype(bf16)

    w1pT = jnp.transpose(W1[:, :PD], (1, 0))
    w1f0T = jnp.transpose(W1[:, PD:C], (1, 0))
    w1f1T = jnp.transpose(W1[:, C:], (1, 0))
    w1a = jnp.concatenate(
        [-w1pT, w1f0T, jnp.zeros((72 - C, 128), f32)]).astype(bf16)
    w1b = jnp.concatenate(
        [w1pT, w1f1T, jnp.zeros((80 - C, 128), f32)]).astype(bf16)
    w2t = jnp.transpose(W2, (1, 0)).astype(bf16)
    w3t = jnp.transpose(W3, (1, 0)).astype(bf16)
    b1r = b1.reshape(1, 128)
    b2r = b2.reshape(1, 128)
    b3r = b3.reshape(1, 256)

    a1 = _a1(pts1p, w1b)

    knns = [_knn(p0pad[b:b + 1], p1pad[b:b + 1], b) for b in range(B)]
    gths = [_sc_gather(a1, idx.reshape(-1)) for idx, _ in knns]
    out = jnp.zeros((B, CO, N), jnp.float32)
    for b in range(B):
        out = _mlp(out, b, gths[b], pts0p[b * N:(b + 1) * N], p0pad[b:b + 1],
                   knns[b][1], w1a, w2t, w3t, b1r, b2r, b3r)
    return out


# speedup vs baseline: 1.0159x; 1.0159x over previous
import functools

import jax
import jax.numpy as jnp
from jax import lax
from jax.experimental import pallas as pl
from jax.experimental.pallas import tpu as pltpu
from jax.experimental.pallas import tpu_sc as plsc

B = 4
N = 2048
C = 67
PD = 3
K = 16
R2 = 4.0
_KEY_R2 = 0x40800000
QB = 256
NB = N // QB
CO = 259


def _knn_body(p0_ref, p1_ref, idx_ref, d2v_ref, *, base):
    p0 = p0_ref[0]
    p1 = p1_ref[0]
    s0 = jnp.sum(p0 * p0, axis=0, keepdims=True)
    s1 = jnp.sum(p1 * p1, axis=1, keepdims=True)
    cross = lax.dot_general(
        p1.astype(jnp.bfloat16), p0.astype(jnp.bfloat16),
        (((1,), (0,)), ((), ())), preferred_element_type=jnp.float32)
    d2 = s1 + s0 - 2.0 * cross
    bits = lax.bitcast_convert_type(d2, jnp.int32)
    iota_j = lax.broadcasted_iota(jnp.int32, (N, QB), 0)
    mkey = bits ^ (jnp.right_shift(bits, 31) & jnp.int32(0x7FFFFFFF))
    key = (mkey & jnp.int32(~2047)) | iota_j
    q = N // 4
    s0, s1, s2, s3 = (key[i * q:(i + 1) * q] for i in range(4))
    s0, s1 = jnp.minimum(s0, s1), jnp.maximum(s0, s1)
    s2, s3 = jnp.minimum(s2, s3), jnp.maximum(s2, s3)
    s0, s2 = jnp.minimum(s0, s2), jnp.maximum(s0, s2)
    s1, s3 = jnp.minimum(s1, s3), jnp.maximum(s1, s3)
    s1, s2 = jnp.minimum(s1, s2), jnp.maximum(s1, s2)
    last = jnp.full((1, QB), jnp.iinfo(jnp.int32).min, jnp.int32)
    big = jnp.iinfo(jnp.int32).max
    for k in range(K):
        cand = jnp.where(s3 > last, s3, big)
        cand = jnp.where(s2 > last, s2, cand)
        cand = jnp.where(s1 > last, s1, cand)
        cand = jnp.where(s0 > last, s0, cand)
        m = jnp.min(cand, axis=0, keepdims=True)
        idx_ref[0, pl.ds(k, 1), :] = (m & jnp.int32(2047)) + base
        d2v_ref[0, pl.ds(k, 1), :] = m
        last = m


def _knn(p0pad, p1pad, b):
    return pl.pallas_call(
        functools.partial(_knn_body, base=b * N),
        grid=(NB,),
        in_specs=[
            pl.BlockSpec((1, 8, QB), lambda i: (0, 0, i)),
            pl.BlockSpec((1, N, 8), lambda i: (0, 0, 0)),
        ],
        out_specs=[
            pl.BlockSpec((1, K, QB), lambda i: (i, 0, 0)),
            pl.BlockSpec((1, K, QB), lambda i: (i, 0, 0)),
        ],
        out_shape=[
            jax.ShapeDtypeStruct((NB, K, QB), jnp.int32),
            jax.ShapeDtypeStruct((NB, K, QB), jnp.int32),
        ],
    )(p0pad, p1pad)


def _a1_body(t_ref, w_ref, o_ref):
    o_ref[...] = jnp.dot(t_ref[...], w_ref[...],
                         preferred_element_type=jnp.float32)


def _a1(table, w1b):
    return pl.pallas_call(
        _a1_body,
        grid=(1,),
        in_specs=[pl.BlockSpec((B * N, 80), lambda i: (0, 0)),
                  pl.BlockSpec((80, 128), lambda i: (0, 0))],
        out_specs=pl.BlockSpec((B * N, 128), lambda i: (0, 0)),
        out_shape=jax.ShapeDtypeStruct((B * N, 128), jnp.float32),
    )(table, w1b)


_NC = 2
_NS = 16
_NW = _NC * _NS
_BTOT = N * K
_BPW = _BTOT // _NW
_GCH = 256


def _sc_gather(table, idxf):
    mesh = plsc.VectorSubcoreMesh(core_axis_name="c", subcore_axis_name="s")

    @functools.partial(
        pl.kernel, mesh=mesh,
        out_type=jax.ShapeDtypeStruct((_BTOT, 128), jnp.float32),
        scratch_types=[
            pltpu.VMEM((_BPW,), jnp.int32),
            pltpu.VMEM((_GCH, 128), jnp.float32),
            pltpu.VMEM((_GCH, 128), jnp.float32),
            pltpu.SemaphoreType.DMA,
            pltpu.SemaphoreType.DMA,
        ])
    def gk(table_hbm, idx_hbm, out_hbm, idx_v, buf0, buf1, sem0, sem1):
        wid = lax.axis_index("s") * _NC + lax.axis_index("c")
        base = wid * _BPW
        pltpu.sync_copy(idx_hbm.at[pl.ds(base, _BPW)], idx_v)
        bufs = (buf0, buf1)
        sems = (sem0, sem1)
        nch = _BPW // _GCH
        for c in range(nch):
            pltpu.make_async_copy(
                table_hbm.at[idx_v.at[pl.ds(c * _GCH, _GCH)]],
                bufs[c % 2], sems[c % 2]).start()
            if c >= 1:
                pltpu.make_async_copy(
                    table_hbm.at[idx_v.at[pl.ds((c - 1) * _GCH, _GCH)]],
                    bufs[(c - 1) % 2], sems[(c - 1) % 2]).wait()
                pltpu.sync_copy(bufs[(c - 1) % 2],
                                out_hbm.at[pl.ds(base + (c - 1) * _GCH, _GCH)])
        pltpu.make_async_copy(
            table_hbm.at[idx_v.at[pl.ds((nch - 1) * _GCH, _GCH)]],
            bufs[(nch - 1) % 2], sems[(nch - 1) % 2]).wait()
        pltpu.sync_copy(bufs[(nch - 1) % 2],
                        out_hbm.at[pl.ds(base + (nch - 1) * _GCH, _GCH)])

    return gk(table, idxf)


def _mlp_body(*refs):
    if len(refs) == 12:
        (prev_ref, gth_ref, pts0_ref, p0_ref, d2v_ref, w1a_ref, w2_ref,
         w3_ref, b1_ref, b2_ref, b3_ref, out_ref) = refs
        del prev_ref
    else:
        (gth_ref, pts0_ref, p0_ref, d2v_ref, w1a_ref, w2_ref,
         w3_ref, b1_ref, b2_ref, b3_ref, out_ref) = refs
    g = gth_ref[...]
    pts0 = pts0_ref[...]
    a0 = jnp.dot(pts0, w1a_ref[...], preferred_element_type=jnp.float32)
    d2v = jnp.transpose(d2v_ref[0], (1, 0))
    b1 = b1_ref[...]
    h1 = jnp.concatenate(
        [jax.nn.relu(g[k * QB:(k + 1) * QB] + a0 + b1) for k in range(K)]
    ).astype(jnp.bfloat16)
    h2 = jax.nn.relu(jnp.dot(h1, w2_ref[...], preferred_element_type=jnp.float32)
                     + b2_ref[...]).astype(jnp.bfloat16)
    h3 = jax.nn.relu(jnp.dot(h2, w3_ref[...], preferred_element_type=jnp.float32)
                     + b3_ref[...])
    acc = jnp.zeros((QB, 256), jnp.float32)
    for k in range(K):
        mask = d2v[:, k:k + 1] >= _KEY_R2
        acc = jnp.maximum(acc, jnp.where(mask, 0.0, h3[k * QB:(k + 1) * QB]))
    out_ref[0, :PD, :] = p0_ref[0, :PD, :]
    out_ref[0, PD:, :] = jnp.transpose(acc, (1, 0))


def _mlp(prev, b, gth, pts0p, p0pad, d2v, w1a, w2t, w3t, b1r, b2r, b3r):
    wspec = lambda shape: pl.BlockSpec(shape, lambda i: (0, 0))
    prev_args = () if prev is None else (prev,)
    prev_specs = [] if prev is None else [pl.BlockSpec(memory_space=pl.ANY)]
    return pl.pallas_call(
        _mlp_body,
        grid=(NB,),
        in_specs=prev_specs + [
            pl.BlockSpec((K * QB, 128), lambda i: (i, 0)),
            pl.BlockSpec((QB, 72), lambda i: (i, 0)),
            pl.BlockSpec((1, 8, QB), lambda i: (0, 0, i)),
            pl.BlockSpec((1, K, QB), lambda i: (i, 0, 0)),
            wspec((72, 128)), wspec((128, 128)),
            wspec((128, 256)), wspec((1, 128)), wspec((1, 128)),
            wspec((1, 256)),
        ],
        out_specs=pl.BlockSpec((1, CO, QB), lambda i, _b=b: (_b, 0, i)),
        out_shape=jax.ShapeDtypeStruct((B, CO, N), jnp.float32),
        input_output_aliases={0: 0} if prev is not None else {},
    )(*prev_args, gth, pts0p, p0pad, d2v, w1a, w2t, w3t, b1r, b2r, b3r)


def kernel(clouds0, clouds1, W1, b1, W2, b2, W3, b3):
    f32 = jnp.float32
    bf16 = jnp.bfloat16
    p0pad = jnp.pad(clouds0[:, :PD, :], ((0, 0), (0, 8 - PD), (0, 0)))
    p1pad = jnp.pad(jnp.transpose(clouds1[:, :PD, :], (0, 2, 1)),
                    ((0, 0), (0, 0), (0, 8 - PD)))

    pts1 = jnp.transpose(clouds1, (0, 2, 1)).reshape(B * N, C)
    pts1p = jnp.pad(pts1, ((0, 0), (0, 80 - C))).astype(bf16)
    pts0 = jnp.transpose(clouds0, (0, 2, 1)).reshape(B * N, C)
    pts0p = jnp.pad(pts0, ((0, 0), (0, 72 - C))).astype(bf16)

    w1pT = jnp.transpose(W1[:, :PD], (1, 0))
    w1f0T = jnp.transpose(W1[:, PD:C], (1, 0))
    w1f1T = jnp.transpose(W1[:, C:], (1, 0))
    w1a = jnp.concatenate(
        [-w1pT, w1f0T, jnp.zeros((72 - C, 128), f32)]).astype(bf16)
    w1b = jnp.concatenate(
        [w1pT, w1f1T, jnp.zeros((80 - C, 128), f32)]).astype(bf16)
    w2t = jnp.transpose(W2, (1, 0)).astype(bf16)
    w3t = jnp.transpose(W3, (1, 0)).astype(bf16)
    b1r = b1.reshape(1, 128)
    b2r = b2.reshape(1, 128)
    b3r = b3.reshape(1, 256)

    a1 = _a1(pts1p, w1b)

    knns = [_knn(p0pad[b:b + 1], p1pad[b:b + 1], b) for b in range(B)]
    gths = [_sc_gather(a1, idx.reshape(-1)) for idx, _ in knns]
    out = None
    for b in range(B):
        out = _mlp(out, b, gths[b], pts0p[b * N:(b + 1) * N], p0pad[b:b + 1],
                   knns[b][1], w1a, w2t, w3t, b1r, b2r, b3r)
    return out
